# baseline (speedup 1.0000x reference)
import functools

import jax
import jax.numpy as jnp
from jax import lax
from jax.experimental import pallas as pl
from jax.experimental.pallas import tpu as pltpu
from jax.experimental.pallas import tpu_sc as plsc

HS = 64
ROWW = HS + 1
LPREF = 200
W = 16
NWIN = 13


def _fm_body(items_hbm, feat_hbm, user_hbm, bias_hbm, ui_hbm, pref_hbm,
             res_out, pbias_out, pnz_out, ui_bias_out, ui_nz_out, parts_hbm,
             pref_v, uiv_v, tiles_v, stage64_v, urow_v, psum_v,
             bstage_v, res_v, uistage_v, parts_v, sem):
    cid = lax.axis_index("c")
    sid = lax.axis_index("s")
    lanes = lax.iota(jnp.int32, W)
    colb = jnp.full((W,), HS, jnp.int32)

    @pl.when(cid == 0)
    def _core0():
        @pl.when(sid < NWIN)
        def _pref_windows():
            start = jnp.minimum(sid * W, LPREF - W)
            start = pl.multiple_of(start, 8)
            pltpu.sync_copy(pref_hbm, pref_v)
            idxvec = plsc.load_gather(pref_v, [start + lanes])
            copies = []
            row_in_tile = []
            for j in range(W):
                sj = jnp.max(jnp.where(lanes == j, idxvec, 0))
                g8 = pl.multiple_of(sj & -8, 8)
                copies.append(pltpu.async_copy(
                    feat_hbm.at[pl.ds(g8, 8)],
                    tiles_v.at[pl.ds(j * 8, 8)], sem))
                row_in_tile.append(jnp.full((W,), j * 8, jnp.int32)
                                   + (sj - g8))
            for c in copies:
                c.wait()
            bstage = jnp.zeros((W,), jnp.float32)
            for j in range(W):
                for k in range(HS // W):
                    v = plsc.load_gather(
                        tiles_v, [row_in_tile[j], lanes + (k * W)])
                    stage64_v[j, pl.ds(k * W, W)] = v
                bj = plsc.load_gather(tiles_v, [row_in_tile[j], colb])
                bstage = jnp.where(lanes == j, bj, bstage)
            bstage_v[...] = bstage
            pltpu.sync_copy(stage64_v, pnz_out.at[pl.ds(start, W)])
            pltpu.sync_copy(bstage_v, pbias_out.at[pl.ds(start, W)])
            thresh = sid * W - start
            for k in range(HS // W):
                acc = jnp.zeros((W,), jnp.float32)
                for r in range(W):
                    v = stage64_v[r, pl.ds(k * W, W)]
                    acc = acc + jnp.where(r >= thresh, v, 0.0)
                psum_v[0, pl.ds(k * W, W)] = acc
            pltpu.sync_copy(psum_v, parts_hbm.at[sid, pl.ds(0, 1)])

        @pl.when(sid == NWIN)
        def _ui_rows():
            pltpu.sync_copy(ui_hbm, uiv_v)
            item_vec = plsc.load_gather(uiv_v, [jnp.full((W,), 1, jnp.int32)])
            si = jnp.max(item_vec)
            g8 = pl.multiple_of(si & -8, 8)
            item_copy = pltpu.async_copy(
                items_hbm.at[pl.ds(g8, 8)], tiles_v.at[pl.ds(0, 8)], sem)
            pltpu.sync_copy(user_hbm, urow_v)
            item_copy.wait()
            rit = jnp.zeros((W,), jnp.int32) + (si - g8)
            for k in range(HS // W):
                uistage_v[0, pl.ds(k * W, W)] = urow_v[0, pl.ds(k * W, W)]
                uistage_v[1, pl.ds(k * W, W)] = plsc.load_gather(
                    tiles_v, [rit, lanes + (k * W)])
            pltpu.sync_copy(uistage_v, ui_nz_out)
            ub = plsc.load_gather(urow_v, [jnp.zeros((W,), jnp.int32), colb])
            ib = plsc.load_gather(tiles_v, [rit, colb])
            bstage_v[...] = jnp.where(lanes == 0, ub, ib)
            pltpu.sync_copy(bstage_v.at[pl.ds(0, 2)], ui_bias_out)

        plsc.subcore_barrier()

        @pl.when(sid == NWIN)
        def _finalize():
            pltpu.sync_copy(parts_hbm, parts_v)
            pltpu.sync_copy(bias_hbm, res_v.at[pl.ds(0, 1)])
            acc = jnp.zeros((W,), jnp.float32)
            for k in range(HS // W):
                p = jnp.zeros((W,), jnp.float32)
                for s in range(NWIN):
                    p = p + parts_v[s, 0, pl.ds(k * W, W)]
                u = uistage_v[0, pl.ds(k * W, W)]
                it = uistage_v[1, pl.ds(k * W, W)]
                acc = acc + u * it + (u + it) * p
            total = jnp.sum(acc)
            bvec = plsc.load_gather(res_v, [jnp.zeros((W,), jnp.int32)])
            res_v[...] = bvec + total
            pltpu.sync_copy(res_v.at[pl.ds(0, 1)], res_out)


@functools.partial(
    pl.kernel,
    out_type=(
        jax.ShapeDtypeStruct((1,), jnp.float32),
        jax.ShapeDtypeStruct((LPREF,), jnp.float32),
        jax.ShapeDtypeStruct((LPREF, HS), jnp.float32),
        jax.ShapeDtypeStruct((2,), jnp.float32),
        jax.ShapeDtypeStruct((2, HS), jnp.float32),
        jax.ShapeDtypeStruct((W, 8, HS), jnp.float32),
    ),
    mesh=plsc.VectorSubcoreMesh(
        core_axis_name="c", subcore_axis_name="s", num_cores=2,
        num_subcores=16),
    compiler_params=pltpu.CompilerParams(needs_layout_passes=False),
    scratch_types=(
        pltpu.VMEM((LPREF,), jnp.int32),
        pltpu.VMEM((2,), jnp.int32),
        pltpu.VMEM((8 * W, ROWW), jnp.float32),
        pltpu.VMEM((W, HS), jnp.float32),
        pltpu.VMEM((1, ROWW), jnp.float32),
        pltpu.VMEM((1, HS), jnp.float32),
        pltpu.VMEM((W,), jnp.float32),
        pltpu.VMEM((W,), jnp.float32),
        pltpu.VMEM((2, HS), jnp.float32),
        pltpu.VMEM((W, 8, HS), jnp.float32),
        pltpu.SemaphoreType.DMA,
    ),
)
def _fm_sc(items_hbm, feat_hbm, user_hbm, bias_hbm, ui_hbm, pref_hbm,
           res_out, pbias_out, pnz_out, ui_bias_out, ui_nz_out, parts_hbm,
           *scratch):
    _fm_body(items_hbm, feat_hbm, user_hbm, bias_hbm, ui_hbm, pref_hbm,
             res_out, pbias_out, pnz_out, ui_bias_out, ui_nz_out, parts_hbm,
             *scratch)


def kernel(items_emb, feature_emb, user_emb, Bias, ui_pair, feature_index,
           preference_index):
    del feature_index
    res, pbias, pnz, uibias, uinz, _ = _fm_sc(
        items_emb, feature_emb, user_emb, Bias,
        ui_pair.reshape(2), preference_index.reshape(LPREF))
    fbias = jnp.concatenate([uibias, pbias]).reshape(1, LPREF + 2, 1)
    nz = jnp.concatenate([uinz, pnz], axis=0).reshape(1, LPREF + 2, HS)
    return (res.reshape(1, 1), fbias, nz)
